# baseline (speedup 1.0000x reference)
import functools

import jax
import jax.numpy as jnp
from jax import lax
from jax.experimental import pallas as pl
from jax.experimental.pallas import tpu as pltpu
from jax.experimental.pallas import tpu_sc as plsc

_D = 64
_L = 50
_NW = 32
_NC = 2
_NEG_BIG = -3.0e38


def _sc_gather_ujp(user_table, item_table, playlist_table,
                   uids, iids, pids, b_sz):
    per_w = b_sz // _NW
    mesh = plsc.VectorSubcoreMesh(core_axis_name="c", subcore_axis_name="s")
    row_t = jax.ShapeDtypeStruct((b_sz, _D), jnp.float32)

    @functools.partial(
        pl.kernel,
        mesh=mesh,
        compiler_params=pltpu.CompilerParams(use_tc_tiling_on_sc=False, needs_layout_passes=False),
        out_type=[row_t, row_t, row_t],
        scratch_types=[
            pltpu.VMEM((per_w,), jnp.int32),
            pltpu.VMEM((per_w,), jnp.int32),
            pltpu.VMEM((per_w,), jnp.int32),
            pltpu.VMEM((per_w, _D), jnp.float32),
            pltpu.VMEM((per_w, _D), jnp.float32),
            pltpu.VMEM((per_w, _D), jnp.float32),
            pltpu.SemaphoreType.DMA,
            pltpu.SemaphoreType.DMA,
            pltpu.SemaphoreType.DMA,
        ],
    )
    def k(ut_hbm, it_hbm, pt_hbm, uid_hbm, iid_hbm, pid_hbm,
          uo_hbm, jo_hbm, po_hbm, uix, iix, pix, ub, jb, pb,
          sem_u, sem_j, sem_p):
        wid = lax.axis_index("s") * _NC + lax.axis_index("c")
        base = wid * per_w
        pltpu.sync_copy(uid_hbm.at[wid], uix)
        pltpu.sync_copy(iid_hbm.at[wid], iix)
        pltpu.sync_copy(pid_hbm.at[wid], pix)
        pltpu.async_copy(ut_hbm.at[uix], ub, sem_u)
        pltpu.async_copy(it_hbm.at[iix], jb, sem_j)
        pltpu.async_copy(pt_hbm.at[pix], pb, sem_p)
        pltpu.make_async_copy(ut_hbm.at[uix], ub, sem_u).wait()
        pltpu.make_async_copy(it_hbm.at[iix], jb, sem_j).wait()
        pltpu.make_async_copy(pt_hbm.at[pix], pb, sem_p).wait()
        pltpu.sync_copy(ub, uo_hbm.at[pl.ds(base, per_w)])
        pltpu.sync_copy(jb, jo_hbm.at[pl.ds(base, per_w)])
        pltpu.sync_copy(pb, po_hbm.at[pl.ds(base, per_w)])

    return k(user_table, item_table, playlist_table, uids, iids, pids)


def _tc_coeffs(u, j, p, W1, b1, W2, b2, A1, A2, b_sz):
    def body(u_ref, j_ref, p_ref, w1_ref, b1_ref, w2_ref, b2_ref,
             a1_ref, a2_ref, g_ref, c_ref, abar_ref):
        uu = u_ref[...]
        jj = j_ref[...]
        pp = p_ref[...]
        w1 = w1_ref[...]
        w2 = w2_ref[...]
        dn = (((1,), (1,)), ((), ()))
        q1 = (lax.dot_general(uu, w1[:, :_D], dn,
                              preferred_element_type=jnp.float32)
              + lax.dot_general(jj, w1[:, _D:], dn,
                                preferred_element_type=jnp.float32)
              + b1_ref[...][None, :])
        q2 = (lax.dot_general(pp, w2[:, :_D], dn,
                              preferred_element_type=jnp.float32)
              + lax.dot_general(jj, w2[:, _D:], dn,
                                preferred_element_type=jnp.float32)
              + b2_ref[...][None, :])
        a1 = a1_ref[...][:, 0] ** 2
        a2 = a2_ref[...][:, 0] ** 2
        g_ref[...] = 2.0 * (q1 * a1[None, :] + q2 * a2[None, :])
        c_ref[...] = jnp.sum(a1[None, :] * q1 * q1
                             + a2[None, :] * q2 * q2, axis=1)
        abar_ref[...] = a1 + a2

    return pl.pallas_call(
        body,
        out_shape=[
            jax.ShapeDtypeStruct((b_sz, _D), jnp.float32),
            jax.ShapeDtypeStruct((b_sz,), jnp.float32),
            jax.ShapeDtypeStruct((_D,), jnp.float32),
        ],
    )(u, j, p, W1, b1, W2, b2, A1, A2)


def _sc_softmax_only(t3, b_sz):
    per_w = b_sz // _NW
    n_grp = (_L + 15) // 16
    mesh = plsc.VectorSubcoreMesh(core_axis_name="c", subcore_axis_name="s")

    @functools.partial(
        pl.kernel,
        mesh=mesh,
        compiler_params=pltpu.CompilerParams(use_tc_tiling_on_sc=False, needs_layout_passes=False),
        out_type=jax.ShapeDtypeStruct((_NW, per_w), jnp.float32),
        scratch_types=[
            pltpu.VMEM((per_w, 64), jnp.float32),
            pltpu.VMEM((per_w,), jnp.float32),
            pltpu.SemaphoreType.DMA,
        ],
    )
    def k(t_hbm, out_hbm, t_v, out_v, sem):
        wid = lax.axis_index("s") * _NC + lax.axis_index("c")
        pltpu.sync_copy(t_hbm.at[wid], t_v)
        lanes = lax.iota(jnp.int32, 16)
        valid = [lanes + 16 * g < _L for g in range(n_grp)]
        lane0 = lanes == 0

        def body(bi, carry):
            brow = jnp.full((16,), bi, jnp.int32)
            tg = [t_v[bi, pl.ds(16 * g, 16)] for g in range(n_grp)]
            ntg = [jnp.where(valid[g], -tg[g], _NEG_BIG)
                   for g in range(n_grp)]
            mx = ntg[0]
            for g in range(1, n_grp):
                mx = jnp.maximum(mx, ntg[g])
            rmax = jnp.max(mx)
            z = jnp.zeros((16,), jnp.float32)
            s = jnp.zeros((16,), jnp.float32)
            for g in range(n_grp):
                e = jnp.where(valid[g], jnp.exp(ntg[g] - rmax),
                              jnp.float32(0.0))
                z = z + e
                s = s + tg[g] * e
            sv = jnp.zeros((16,), jnp.float32) + jnp.sum(s)
            zv = jnp.zeros((16,), jnp.float32) + jnp.sum(z)
            plsc.store_scatter(out_v, [brow], sv / zv, mask=lane0)
            return carry

        lax.fori_loop(0, per_w, body, 0)
        pltpu.sync_copy(out_v, out_hbm.at[wid])

    return k(t3)


def _sc_memory_attention(item_table, item_biases, idsp, mask3, g3, c2, abar,
                         b_sz, debug_t_out=False):
    per_w = b_sz // _NW
    n_pair = per_w // 2
    n_grp = (_L + 15) // 16
    mesh = plsc.VectorSubcoreMesh(core_axis_name="c", subcore_axis_name="s")
    out_t = (jax.ShapeDtypeStruct((_NW, per_w, 64), jnp.float32)
             if debug_t_out else
             jax.ShapeDtypeStruct((_NW, per_w), jnp.float32))

    @functools.partial(
        pl.kernel,
        mesh=mesh,
        compiler_params=pltpu.CompilerParams(use_tc_tiling_on_sc=False, needs_layout_passes=False),
        out_type=out_t,
        scratch_types=[
            pltpu.VMEM((per_w, 64), jnp.float32),
            pltpu.VMEM((n_pair, 112), jnp.int32),
            pltpu.VMEM((2, 112), jnp.int32),
            pltpu.VMEM((per_w, _L), jnp.float32),
            pltpu.VMEM((per_w, _D), jnp.float32),
            pltpu.VMEM((per_w,), jnp.float32),
            pltpu.VMEM((_D,), jnp.float32),
            pltpu.VMEM((104, _D), jnp.float32),
            pltpu.VMEM((104, _D), jnp.float32),
            pltpu.VMEM((104, 16), jnp.float32),
            pltpu.VMEM((104, 16), jnp.float32),
            pltpu.VMEM((per_w,), jnp.float32),
            pltpu.VMEM((64, 16), jnp.float32),
            pltpu.SemaphoreType.DMA,
            pltpu.SemaphoreType.DMA,
            pltpu.SemaphoreType.DMA,
            pltpu.SemaphoreType.DMA,
        ],
    )
    def k(tbl_hbm, bias_hbm, ids_hbm, mask_hbm, g_hbm, c_hbm, abar_hbm,
          out_hbm, t_stage, ids_v, idshi_v, mask_v, g_v, c_v, abar_v,
          m0, m1, bb0, bb1, out_v, part_v, msem0, msem1, bsem0, bsem1):
        msems = (msem0, msem1)
        bsems = (bsem0, bsem1)
        wid = lax.axis_index("s") * _NC + lax.axis_index("c")
        pltpu.sync_copy(ids_hbm.at[wid], ids_v)
        pltpu.sync_copy(mask_hbm.at[wid], mask_v)
        pltpu.sync_copy(g_hbm.at[wid], g_v)
        pltpu.sync_copy(c_hbm.at[wid], c_v)
        pltpu.sync_copy(abar_hbm, abar_v)

        mbufs = (m0, m1)
        bbufs = (bb0, bb1)
        lanes = lax.iota(jnp.int32, 16)
        base_idx = [jnp.minimum(lanes + 16 * g, _L - 1) for g in range(n_grp)]
        valid = [lanes + 16 * g < _L for g in range(n_grp)]
        zcol = jnp.zeros((16,), jnp.int32)
        lane0 = lanes == 0

        def pair_idx(pr):
            return ids_v.at[pr, pl.ds(0, 104)]

        def stage_idshi(pr, parity):
            for kk in range(7):
                idshi_v[parity, pl.ds(16 * kk, 16)] = lax.shift_right_logical(
                    ids_v[pr, pl.ds(16 * kk, 16)], 4)

        def bias_idx(parity):
            return idshi_v.at[parity, pl.ds(0, 104)]

        for _r in range(_L - 48, 16):
            part_v[48 + _r, pl.ds(0, 16)] = jnp.zeros((16,), jnp.float32)

        stage_idshi(0, 0)
        pltpu.async_copy(tbl_hbm.at[pair_idx(0)], m0, msem0)
        pltpu.async_copy(bias_hbm.at[bias_idx(0)], bb0, bsem0)

        def outer(io, carry):
            for kbuf in range(2):
                pr = io * 2 + kbuf
                mb = mbufs[kbuf]
                bb = bbufs[kbuf]
                nxt = pr + 1

                @pl.when(nxt < n_pair)
                def _():
                    stage_idshi(nxt, 1 - kbuf)
                    pltpu.async_copy(tbl_hbm.at[pair_idx(nxt)],
                                     mbufs[1 - kbuf], msems[1 - kbuf])
                    pltpu.async_copy(bias_hbm.at[bias_idx(1 - kbuf)],
                                     bbufs[1 - kbuf], bsems[1 - kbuf])

                pltpu.make_async_copy(
                    tbl_hbm.at[pair_idx(pr)], mb, msems[kbuf]).wait()
                pltpu.make_async_copy(
                    bias_hbm.at[bias_idx(kbuf)], bb, bsems[kbuf]).wait()

                for sub in range(2):
                    bi = pr * 2 + sub
                    rbase = sub * _L
                    row_idx = [base_idx[g] + rbase for g in range(n_grp)]

                    g_c = [g_v[bi, pl.ds(16 * c, 16)] for c in range(4)]
                    a_c = [abar_v[pl.ds(16 * c, 16)] for c in range(4)]

                    def row_partial(row, mb=mb, g_c=g_c, a_c=a_c):
                        rs = jnp.full((16,), row, jnp.int32)
                        acc = None
                        for c in range(4):
                            mv = plsc.load_gather(mb, [rs, lanes + 16 * c])
                            tmp = mv * a_c[c] - g_c[c]
                            acc = mv * tmp if acc is None else acc + mv * tmp
                        return acc

                    def rbody(r0, carry, rbase=rbase,
                              row_partial=row_partial):
                        for dr in range(4):
                            r = r0 * 4 + dr
                            for g in range(3):
                                plsc.store_scatter(
                                    part_v,
                                    [jnp.full((16,), 16 * g, jnp.int32) + r,
                                     lanes],
                                    row_partial(rbase + 16 * g + r))
                        return carry

                    lax.fori_loop(0, 4, rbody, 0)
                    for r in range(_L - 48):
                        part_v[48 + r, pl.ds(0, 16)] = row_partial(
                            rbase + 48 + r)

                    accs = []
                    for g in range(n_grp):
                        tr = None
                        for c in range(16):
                            cs = jnp.full((16,), c, jnp.int32)
                            pv = plsc.load_gather(
                                part_v, [lanes + 16 * g, cs])
                            tr = pv if tr is None else tr + pv
                        accs.append(tr)

                    brow = jnp.full((16,), bi, jnp.int32)
                    cbv = plsc.load_gather(c_v, [brow])
                    tg = []
                    ntg = []
                    mk_dbg = []
                    prsplat = jnp.full((16,), pr, jnp.int32)
                    for g in range(n_grp):
                        idvec = plsc.load_gather(ids_v, [prsplat, row_idx[g]])
                        bv = plsc.load_gather(bb, [row_idx[g], idvec & 15])
                        mk = plsc.load_gather(mask_v, [brow, base_idx[g]])
                        mk_dbg.append(mk)
                        t = (cbv + accs[g] + 2.0 * bv) * mk
                        tg.append(t)
                        ntg.append(jnp.where(valid[g], -t, _NEG_BIG))
                    if debug_t_out:
                        for g in range(n_grp):
                            t_stage[bi, pl.ds(16 * g, 16)] = tg[g]
                        continue
                    mx = ntg[0]
                    for g in range(1, n_grp):
                        mx = jnp.maximum(mx, ntg[g])
                    rmax = jnp.max(mx)
                    z = jnp.zeros((16,), jnp.float32)
                    s = jnp.zeros((16,), jnp.float32)
                    for g in range(n_grp):
                        e = jnp.where(valid[g], jnp.exp(ntg[g] - rmax),
                                      jnp.float32(0.0))
                        z = z + e
                        s = s + jnp.where(valid[g], tg[g] * e,
                                          jnp.float32(0.0))
                    sv = jnp.zeros((16,), jnp.float32) + jnp.sum(s)
                    zv = jnp.zeros((16,), jnp.float32) + jnp.sum(z)
                    plsc.store_scatter(out_v, [brow], sv / zv, mask=lane0)
            return carry

        lax.fori_loop(0, n_pair // 2, outer, 0)
        if debug_t_out:
            pltpu.sync_copy(t_stage, out_hbm.at[wid])
        else:
            pltpu.sync_copy(out_v, out_hbm.at[wid])

    bias16 = item_biases.reshape(-1, 16)
    return k(item_table, bias16, idsp, mask3, g3, c2, abar)


def kernel(user_ids, item_ids, playlist_ids, seq_item_ids, mask,
           user_table, item_table, playlist_table, item_biases,
           W1, b1, W2, b2, A1, A2):
    b_sz, l_sz = seq_item_ids.shape
    per_w = b_sz // _NW

    uids = user_ids.reshape(_NW, per_w).astype(jnp.int32)
    iids = item_ids.reshape(_NW, per_w).astype(jnp.int32)
    pids = playlist_ids.reshape(_NW, per_w).astype(jnp.int32)
    u, j, p = _sc_gather_ujp(user_table, item_table, playlist_table,
                             uids, iids, pids, b_sz)

    g, c, abar = _tc_coeffs(u, j, p, W1, b1, W2, b2, A1, A2, b_sz)

    _DEBUG_MODE = "off"
    if _DEBUG_MODE in ("xla_tail", "sc_softmax"):
        m = jnp.take(item_table, seq_item_ids, axis=0)
        biases = jnp.take(item_biases, seq_item_ids, axis=0)[..., 0]
        t = (c[:, None]
             + jnp.einsum('bld,bld->bl', m, m * abar[None, None, :])
             - jnp.einsum('bld,bd->bl', m, g)
             + 2.0 * biases) * mask
        if _DEBUG_MODE == "sc_softmax":
            t3 = jnp.pad(t, ((0, 0), (0, 64 - l_sz))).reshape(
                _NW, per_w, 64)
            return _sc_softmax_only(t3, b_sz).reshape(b_sz)
        weights = jax.nn.softmax(-t, axis=1)
        return jnp.sum(t * weights, axis=1)

    if _DEBUG_MODE == "sc_dloop":
        ids3 = jnp.pad(
            seq_item_ids.astype(jnp.int32).reshape(b_sz // 2, 2 * l_sz),
            ((0, 0), (0, 12))).reshape(_NW, per_w // 2, 112)
        mask3 = mask.reshape(_NW, per_w, l_sz)
        g3 = g.reshape(_NW, per_w, _D)
        c2 = c.reshape(_NW, per_w)
        t3 = _sc_memory_attention(item_table, item_biases, ids3, mask3,
                                   g3, c2, abar, b_sz, debug_t_out=True)
        t = t3.reshape(b_sz, 64)[:, :l_sz]
        weights = jax.nn.softmax(-t, axis=1)
        return jnp.sum(t * weights, axis=1)

    ids3 = jnp.pad(
        seq_item_ids.astype(jnp.int32).reshape(b_sz // 2, 2 * l_sz),
        ((0, 0), (0, 12))).reshape(_NW, per_w // 2, 112)
    mask3 = mask.reshape(_NW, per_w, l_sz)
    g3 = g.reshape(_NW, per_w, _D)
    c2 = c.reshape(_NW, per_w)
    pred = _sc_memory_attention(item_table, item_biases, ids3, mask3,
                                g3, c2, abar, b_sz)
    return pred.reshape(b_sz)


# speedup vs baseline: 1.2516x; 1.2516x over previous
import functools

import jax
import jax.numpy as jnp
from jax import lax
from jax.experimental import pallas as pl
from jax.experimental.pallas import tpu as pltpu
from jax.experimental.pallas import tpu_sc as plsc

_D = 64
_L = 50
_NW = 32
_NC = 2
_NEG_BIG = -3.0e38


def _sc_gather_j(item_table, iids, b_sz):
    per_w = b_sz // _NW
    mesh = plsc.VectorSubcoreMesh(core_axis_name="c", subcore_axis_name="s")

    @functools.partial(
        pl.kernel,
        mesh=mesh,
        compiler_params=pltpu.CompilerParams(use_tc_tiling_on_sc=False, needs_layout_passes=False),
        out_type=jax.ShapeDtypeStruct((b_sz, _D), jnp.float32),
        scratch_types=[
            pltpu.VMEM((per_w,), jnp.int32),
            pltpu.VMEM((per_w, _D), jnp.float32),
            pltpu.SemaphoreType.DMA,
        ],
    )
    def k(it_hbm, iid_hbm, jo_hbm, iix, jb, sem_j):
        wid = lax.axis_index("s") * _NC + lax.axis_index("c")
        base = wid * per_w
        pltpu.sync_copy(iid_hbm.at[wid], iix)
        pltpu.async_copy(it_hbm.at[iix], jb, sem_j).wait()
        pltpu.sync_copy(jb, jo_hbm.at[pl.ds(base, per_w)])

    return k(item_table, iids)


def _tc_coeffs(u, j, p, W1, b1, W2, b2, A1, A2, b_sz):
    def body(u_ref, j_ref, p_ref, w1_ref, b1_ref, w2_ref, b2_ref,
             a1_ref, a2_ref, g_ref, c_ref, abar_ref):
        uu = u_ref[...]
        jj = j_ref[...]
        pp = p_ref[...]
        w1 = w1_ref[...]
        w2 = w2_ref[...]
        dn = (((1,), (1,)), ((), ()))
        q1 = (lax.dot_general(uu, w1[:, :_D], dn,
                              preferred_element_type=jnp.float32)
              + lax.dot_general(jj, w1[:, _D:], dn,
                                preferred_element_type=jnp.float32)
              + b1_ref[...][None, :])
        q2 = (lax.dot_general(pp, w2[:, :_D], dn,
                              preferred_element_type=jnp.float32)
              + lax.dot_general(jj, w2[:, _D:], dn,
                                preferred_element_type=jnp.float32)
              + b2_ref[...][None, :])
        a1 = a1_ref[...][:, 0] ** 2
        a2 = a2_ref[...][:, 0] ** 2
        g_ref[...] = 2.0 * (q1 * a1[None, :] + q2 * a2[None, :])
        c_ref[...] = jnp.sum(a1[None, :] * q1 * q1
                             + a2[None, :] * q2 * q2, axis=1)
        abar_ref[...] = a1 + a2

    return pl.pallas_call(
        body,
        out_shape=[
            jax.ShapeDtypeStruct((b_sz, _D), jnp.float32),
            jax.ShapeDtypeStruct((b_sz,), jnp.float32),
            jax.ShapeDtypeStruct((_D,), jnp.float32),
        ],
    )(u, j, p, W1, b1, W2, b2, A1, A2)


def _sc_softmax_only(t3, b_sz):
    per_w = b_sz // _NW
    n_grp = (_L + 15) // 16
    mesh = plsc.VectorSubcoreMesh(core_axis_name="c", subcore_axis_name="s")

    @functools.partial(
        pl.kernel,
        mesh=mesh,
        compiler_params=pltpu.CompilerParams(use_tc_tiling_on_sc=False, needs_layout_passes=False),
        out_type=jax.ShapeDtypeStruct((_NW, per_w), jnp.float32),
        scratch_types=[
            pltpu.VMEM((per_w, 64), jnp.float32),
            pltpu.VMEM((per_w,), jnp.float32),
            pltpu.SemaphoreType.DMA,
        ],
    )
    def k(t_hbm, out_hbm, t_v, out_v, sem):
        wid = lax.axis_index("s") * _NC + lax.axis_index("c")
        pltpu.sync_copy(t_hbm.at[wid], t_v)
        lanes = lax.iota(jnp.int32, 16)
        valid = [lanes + 16 * g < _L for g in range(n_grp)]
        lane0 = lanes == 0

        def body(bi, carry):
            brow = jnp.full((16,), bi, jnp.int32)
            tg = [t_v[bi, pl.ds(16 * g, 16)] for g in range(n_grp)]
            ntg = [jnp.where(valid[g], -tg[g], _NEG_BIG)
                   for g in range(n_grp)]
            mx = ntg[0]
            for g in range(1, n_grp):
                mx = jnp.maximum(mx, ntg[g])
            rmax = jnp.max(mx)
            z = jnp.zeros((16,), jnp.float32)
            s = jnp.zeros((16,), jnp.float32)
            for g in range(n_grp):
                e = jnp.where(valid[g], jnp.exp(ntg[g] - rmax),
                              jnp.float32(0.0))
                z = z + e
                s = s + tg[g] * e
            sv = jnp.zeros((16,), jnp.float32) + jnp.sum(s)
            zv = jnp.zeros((16,), jnp.float32) + jnp.sum(z)
            plsc.store_scatter(out_v, [brow], sv / zv, mask=lane0)
            return carry

        lax.fori_loop(0, per_w, body, 0)
        pltpu.sync_copy(out_v, out_hbm.at[wid])

    return k(t3)


def _sc_memory_attention(item_table, item_biases, idsp, mask3, g3, c2, abar,
                         b_sz, debug_t_out=False):
    per_w = b_sz // _NW
    n_pair = per_w // 2
    n_grp = (_L + 15) // 16
    mesh = plsc.VectorSubcoreMesh(core_axis_name="c", subcore_axis_name="s")
    out_t = (jax.ShapeDtypeStruct((_NW, per_w, 64), jnp.float32)
             if debug_t_out else
             jax.ShapeDtypeStruct((_NW, per_w), jnp.float32))

    @functools.partial(
        pl.kernel,
        mesh=mesh,
        compiler_params=pltpu.CompilerParams(use_tc_tiling_on_sc=False, needs_layout_passes=False),
        out_type=out_t,
        scratch_types=[
            pltpu.VMEM((per_w, 64), jnp.float32),
            pltpu.VMEM((n_pair, 112), jnp.int32),
            pltpu.VMEM((2, 112), jnp.int32),
            pltpu.VMEM((per_w, _L), jnp.float32),
            pltpu.VMEM((per_w, _D), jnp.float32),
            pltpu.VMEM((per_w,), jnp.float32),
            pltpu.VMEM((_D,), jnp.float32),
            pltpu.VMEM((104, _D), jnp.float32),
            pltpu.VMEM((104, _D), jnp.float32),
            pltpu.VMEM((104, 16), jnp.float32),
            pltpu.VMEM((104, 16), jnp.float32),
            pltpu.VMEM((per_w,), jnp.float32),
            pltpu.VMEM((64, 16), jnp.float32),
            pltpu.SemaphoreType.DMA,
            pltpu.SemaphoreType.DMA,
            pltpu.SemaphoreType.DMA,
            pltpu.SemaphoreType.DMA,
        ],
    )
    def k(tbl_hbm, bias_hbm, ids_hbm, mask_hbm, g_hbm, c_hbm, abar_hbm,
          out_hbm, t_stage, ids_v, idshi_v, mask_v, g_v, c_v, abar_v,
          m0, m1, bb0, bb1, out_v, part_v, msem0, msem1, bsem0, bsem1):
        msems = (msem0, msem1)
        bsems = (bsem0, bsem1)
        wid = lax.axis_index("s") * _NC + lax.axis_index("c")
        pltpu.sync_copy(ids_hbm.at[wid], ids_v)
        pltpu.sync_copy(mask_hbm.at[wid], mask_v)
        pltpu.sync_copy(g_hbm.at[wid], g_v)
        pltpu.sync_copy(c_hbm.at[wid], c_v)
        pltpu.sync_copy(abar_hbm, abar_v)

        mbufs = (m0, m1)
        bbufs = (bb0, bb1)
        lanes = lax.iota(jnp.int32, 16)
        base_idx = [jnp.minimum(lanes + 16 * g, _L - 1) for g in range(n_grp)]
        valid = [lanes + 16 * g < _L for g in range(n_grp)]
        zcol = jnp.zeros((16,), jnp.int32)
        lane0 = lanes == 0

        def pair_idx(pr):
            return ids_v.at[pr, pl.ds(0, 104)]

        def stage_idshi(pr, parity):
            for kk in range(7):
                idshi_v[parity, pl.ds(16 * kk, 16)] = lax.shift_right_logical(
                    ids_v[pr, pl.ds(16 * kk, 16)], 4)

        def bias_idx(parity):
            return idshi_v.at[parity, pl.ds(0, 104)]

        for _r in range(_L - 48, 16):
            part_v[48 + _r, pl.ds(0, 16)] = jnp.zeros((16,), jnp.float32)

        stage_idshi(0, 0)
        pltpu.async_copy(tbl_hbm.at[pair_idx(0)], m0, msem0)
        pltpu.async_copy(bias_hbm.at[bias_idx(0)], bb0, bsem0)

        def outer(io, carry):
            for kbuf in range(2):
                pr = io * 2 + kbuf
                mb = mbufs[kbuf]
                bb = bbufs[kbuf]
                nxt = pr + 1

                @pl.when(nxt < n_pair)
                def _():
                    stage_idshi(nxt, 1 - kbuf)
                    pltpu.async_copy(tbl_hbm.at[pair_idx(nxt)],
                                     mbufs[1 - kbuf], msems[1 - kbuf])
                    pltpu.async_copy(bias_hbm.at[bias_idx(1 - kbuf)],
                                     bbufs[1 - kbuf], bsems[1 - kbuf])

                pltpu.make_async_copy(
                    tbl_hbm.at[pair_idx(pr)], mb, msems[kbuf]).wait()
                pltpu.make_async_copy(
                    bias_hbm.at[bias_idx(kbuf)], bb, bsems[kbuf]).wait()

                for sub in range(2):
                    bi = pr * 2 + sub
                    rbase = sub * _L
                    row_idx = [base_idx[g] + rbase for g in range(n_grp)]

                    g_c = [g_v[bi, pl.ds(16 * c, 16)] for c in range(4)]
                    a_c = [abar_v[pl.ds(16 * c, 16)] for c in range(4)]

                    def row_partial(row, mb=mb, g_c=g_c, a_c=a_c):
                        rs = jnp.full((16,), row, jnp.int32)
                        acc = None
                        for c in range(4):
                            mv = plsc.load_gather(mb, [rs, lanes + 16 * c])
                            tmp = mv * a_c[c] - g_c[c]
                            acc = mv * tmp if acc is None else acc + mv * tmp
                        return acc

                    def rbody(r0, carry, rbase=rbase,
                              row_partial=row_partial):
                        for dr in range(4):
                            r = r0 * 4 + dr
                            for g in range(3):
                                plsc.store_scatter(
                                    part_v,
                                    [jnp.full((16,), 16 * g, jnp.int32) + r,
                                     lanes],
                                    row_partial(rbase + 16 * g + r))
                        return carry

                    lax.fori_loop(0, 4, rbody, 0)
                    for r in range(_L - 48):
                        part_v[48 + r, pl.ds(0, 16)] = row_partial(
                            rbase + 48 + r)

                    accs = []
                    for g in range(n_grp):
                        tr = None
                        for c in range(16):
                            cs = jnp.full((16,), c, jnp.int32)
                            pv = plsc.load_gather(
                                part_v, [lanes + 16 * g, cs])
                            tr = pv if tr is None else tr + pv
                        accs.append(tr)

                    brow = jnp.full((16,), bi, jnp.int32)
                    cbv = plsc.load_gather(c_v, [brow])
                    tg = []
                    ntg = []
                    mk_dbg = []
                    prsplat = jnp.full((16,), pr, jnp.int32)
                    for g in range(n_grp):
                        idvec = plsc.load_gather(ids_v, [prsplat, row_idx[g]])
                        bv = plsc.load_gather(bb, [row_idx[g], idvec & 15])
                        mk = plsc.load_gather(mask_v, [brow, base_idx[g]])
                        mk_dbg.append(mk)
                        t = (cbv + accs[g] + 2.0 * bv) * mk
                        tg.append(t)
                        ntg.append(jnp.where(valid[g], -t, _NEG_BIG))
                    if debug_t_out:
                        for g in range(n_grp):
                            t_stage[bi, pl.ds(16 * g, 16)] = tg[g]
                        continue
                    mx = ntg[0]
                    for g in range(1, n_grp):
                        mx = jnp.maximum(mx, ntg[g])
                    rmax = jnp.max(mx)
                    z = jnp.zeros((16,), jnp.float32)
                    s = jnp.zeros((16,), jnp.float32)
                    for g in range(n_grp):
                        e = jnp.where(valid[g], jnp.exp(ntg[g] - rmax),
                                      jnp.float32(0.0))
                        z = z + e
                        s = s + jnp.where(valid[g], tg[g] * e,
                                          jnp.float32(0.0))
                    sv = jnp.zeros((16,), jnp.float32) + jnp.sum(s)
                    zv = jnp.zeros((16,), jnp.float32) + jnp.sum(z)
                    plsc.store_scatter(out_v, [brow], sv / zv, mask=lane0)
            return carry

        lax.fori_loop(0, n_pair // 2, outer, 0)
        if debug_t_out:
            pltpu.sync_copy(t_stage, out_hbm.at[wid])
        else:
            pltpu.sync_copy(out_v, out_hbm.at[wid])

    bias16 = item_biases.reshape(-1, 16)
    return k(item_table, bias16, idsp, mask3, g3, c2, abar)


def kernel(user_ids, item_ids, playlist_ids, seq_item_ids, mask,
           user_table, item_table, playlist_table, item_biases,
           W1, b1, W2, b2, A1, A2):
    b_sz, l_sz = seq_item_ids.shape
    per_w = b_sz // _NW

    iids = item_ids.reshape(_NW, per_w).astype(jnp.int32)
    u = jnp.take(user_table, user_ids, axis=0)
    p = jnp.take(playlist_table, playlist_ids, axis=0)
    j = _sc_gather_j(item_table, iids, b_sz)

    g, c, abar = _tc_coeffs(u, j, p, W1, b1, W2, b2, A1, A2, b_sz)

    _DEBUG_MODE = "off"
    if _DEBUG_MODE in ("xla_tail", "sc_softmax"):
        m = jnp.take(item_table, seq_item_ids, axis=0)
        biases = jnp.take(item_biases, seq_item_ids, axis=0)[..., 0]
        t = (c[:, None]
             + jnp.einsum('bld,bld->bl', m, m * abar[None, None, :])
             - jnp.einsum('bld,bd->bl', m, g)
             + 2.0 * biases) * mask
        if _DEBUG_MODE == "sc_softmax":
            t3 = jnp.pad(t, ((0, 0), (0, 64 - l_sz))).reshape(
                _NW, per_w, 64)
            return _sc_softmax_only(t3, b_sz).reshape(b_sz)
        weights = jax.nn.softmax(-t, axis=1)
        return jnp.sum(t * weights, axis=1)

    if _DEBUG_MODE == "sc_dloop":
        ids3 = jnp.pad(
            seq_item_ids.astype(jnp.int32).reshape(b_sz // 2, 2 * l_sz),
            ((0, 0), (0, 12))).reshape(_NW, per_w // 2, 112)
        mask3 = mask.reshape(_NW, per_w, l_sz)
        g3 = g.reshape(_NW, per_w, _D)
        c2 = c.reshape(_NW, per_w)
        t3 = _sc_memory_attention(item_table, item_biases, ids3, mask3,
                                   g3, c2, abar, b_sz, debug_t_out=True)
        t = t3.reshape(b_sz, 64)[:, :l_sz]
        weights = jax.nn.softmax(-t, axis=1)
        return jnp.sum(t * weights, axis=1)

    ids3 = jnp.pad(
        seq_item_ids.astype(jnp.int32).reshape(b_sz // 2, 2 * l_sz),
        ((0, 0), (0, 12))).reshape(_NW, per_w // 2, 112)
    mask3 = mask.reshape(_NW, per_w, l_sz)
    g3 = g.reshape(_NW, per_w, _D)
    c2 = c.reshape(_NW, per_w)
    pred = _sc_memory_attention(item_table, item_biases, ids3, mask3,
                                g3, c2, abar, b_sz)
    return pred.reshape(b_sz)
